# baseline (speedup 1.0000x reference)
import numpy as np
import jax
import jax.numpy as jnp
from jax.experimental import pallas as pl
from jax.experimental.pallas import tpu as pltpu

_H, _D = 16, 64
_C, _ITERS, _BITS = 100, 10, 32
_CP = 128
_RB = 256


def _q_kernel(x_ref, wq_ref, bq_ref, q_ref):
    q_ref[...] = jnp.dot(x_ref[...], wq_ref[...]) + bq_ref[...]


def _attn_kernel(q_ref, x_ref, wk_ref, bk_ref, wv_ref, bv_ref,
                 hp_ref, init_ref, wo_ref, bo_ref, out_ref):
    h = pl.program_id(1)
    S = x_ref.shape[1]
    temp = 1.0 / np.sqrt(_D)

    xb = x_ref[0]
    q = q_ref[0, 0]
    k = jnp.dot(xb, wk_ref[0]) + bk_ref[0]
    v = jnp.dot(xb, wv_ref[0]) + bv_ref[0]

    proj = jnp.dot(q, hp_ref[...])
    codes = jnp.where(proj > 0, 1.0, -1.0).astype(jnp.float32)

    c_iota = jax.lax.broadcasted_iota(jnp.int32, (S, _CP), 1)
    pad_mask = jnp.where(c_iota >= _C, 1e30, 0.0)
    s_iota = jax.lax.broadcasted_iota(jnp.int32, (_CP, S), 1)
    sel = (s_iota == init_ref[...]).astype(jnp.float32)
    cent0 = jax.lax.dot_general(sel, codes, (((1,), (0,)), ((), ())))

    ones_s = jnp.ones((S, 1), jnp.float32)
    code_sq = jnp.sum(codes * codes, axis=-1, keepdims=True)

    def _one_hot(cent):
        m = jax.lax.dot_general(codes, cent, (((1,), (1,)), ((), ())))
        centsq = jnp.sum(cent * cent, axis=-1, keepdims=True).T
        d = code_sq - 2.0 * m + centsq + pad_mask
        dmin = jnp.min(d, axis=-1, keepdims=True)
        am = jnp.where(d == dmin, c_iota, _CP)
        assign = jnp.min(am, axis=-1, keepdims=True)
        return (c_iota == assign).astype(jnp.float32)

    def _body(_, cent):
        oh = _one_hot(cent)
        cnt = jnp.maximum(
            jax.lax.dot_general(oh, ones_s, (((0,), (0,)), ((), ()))), 1.0)
        return jax.lax.dot_general(oh, codes, (((0,), (0,)), ((), ()))) / cnt

    cent = jax.lax.fori_loop(0, _ITERS - 1, _body, cent0)
    oh = _one_hot(cent)

    cnt = jnp.maximum(
        jax.lax.dot_general(oh, ones_s, (((0,), (0,)), ((), ()))), 1.0)
    qc = jax.lax.dot_general(oh, q, (((0,), (0,)), ((), ()))) / cnt
    logits = temp * jax.lax.dot_general(qc, k, (((1,), (1,)), ((), ())))
    mx = jnp.max(logits, axis=-1, keepdims=True)
    e = jnp.exp(logits - mx)
    a = e / jnp.sum(e, axis=-1, keepdims=True)
    oc = jax.lax.dot_general(a, v, (((1,), (0,)), ((), ())))
    out_g = jnp.dot(oh, oc)
    contrib = jnp.dot(out_g, wo_ref[0])

    @pl.when(h == 0)
    def _():
        out_ref[0] = contrib + bo_ref[...]

    @pl.when(h != 0)
    def _():
        out_ref[0] = out_ref[0] + contrib


def kernel(x, attention_mask, Wq, bq, Wk, bk, Wv, bv, Wo, bo, hash_planes):
    del attention_mask
    Bx, Sx, Ex = x.shape
    HD = _H * _D
    M = Bx * Sx

    init_idx = np.full((_CP, 1), -1, np.int32)
    init_idx[:_C, 0] = np.linspace(0, Sx - 1, _C).astype(np.int32)
    init_idx = jnp.asarray(init_idx)

    q_flat = pl.pallas_call(
        _q_kernel,
        grid=(M // _RB,),
        in_specs=[
            pl.BlockSpec((_RB, Ex), lambda i: (i, 0)),
            pl.BlockSpec((Ex, HD), lambda i: (0, 0)),
            pl.BlockSpec((1, HD), lambda i: (0, 0)),
        ],
        out_specs=pl.BlockSpec((_RB, HD), lambda i: (i, 0)),
        out_shape=jax.ShapeDtypeStruct((M, HD), jnp.float32),
        compiler_params=pltpu.CompilerParams(
            dimension_semantics=("arbitrary",)),
        interpret=False,
    )(x.reshape(M, Ex), Wq, bq.reshape(1, HD))

    q4 = q_flat.reshape(Bx, Sx, _H, _D).transpose(0, 2, 1, 3)

    Wk3 = Wk.reshape(Ex, _H, _D).transpose(1, 0, 2)
    Wv3 = Wv.reshape(Ex, _H, _D).transpose(1, 0, 2)
    Wo3 = Wo.reshape(_H, _D, Ex)
    bk3 = bk.reshape(_H, 1, _D)
    bv3 = bv.reshape(_H, 1, _D)
    bo2 = bo.reshape(1, Ex)

    out = pl.pallas_call(
        _attn_kernel,
        grid=(Bx, _H),
        in_specs=[
            pl.BlockSpec((1, 1, Sx, _D), lambda b, h: (b, h, 0, 0)),
            pl.BlockSpec((1, Sx, Ex), lambda b, h: (b, 0, 0)),
            pl.BlockSpec((1, Ex, _D), lambda b, h: (h, 0, 0)),
            pl.BlockSpec((1, 1, _D), lambda b, h: (h, 0, 0)),
            pl.BlockSpec((1, Ex, _D), lambda b, h: (h, 0, 0)),
            pl.BlockSpec((1, 1, _D), lambda b, h: (h, 0, 0)),
            pl.BlockSpec((_D, _BITS), lambda b, h: (0, 0)),
            pl.BlockSpec((_CP, 1), lambda b, h: (0, 0)),
            pl.BlockSpec((1, _D, Ex), lambda b, h: (h, 0, 0)),
            pl.BlockSpec((1, Ex), lambda b, h: (0, 0)),
        ],
        out_specs=pl.BlockSpec((1, Sx, Ex), lambda b, h: (b, 0, 0)),
        out_shape=jax.ShapeDtypeStruct((Bx, Sx, Ex), jnp.float32),
        compiler_params=pltpu.CompilerParams(
            dimension_semantics=("parallel", "arbitrary")),
        interpret=False,
    )(q4, x, Wk3, bk3, Wv3, bv3, hash_planes, init_idx, Wo3, bo2)
    return out
